# baseline (speedup 1.0000x reference)
import jax
import jax.numpy as jnp
from jax.experimental import pallas as pl

N_AGENTS = 8
N_ACTIONS = 16
STATE_DIM = 256
EMBED = 64
SAMPLE = 16
BLK = 256


def _mixer_kernel(sT_ref, raT_ref, wq_ref, aqT_ref, t_ref,
                  hw1_w1T_ref, hw1_b1_ref, hw1_w2T_ref, hw1_b2_ref,
                  hwf_w1T_ref, hwf_b1_ref, hwf_w2T_ref, hwf_b2_ref,
                  hb1_wT_ref, hb1_b_ref, v_w1T_ref, v_b1_ref, v_w2_ref, v_b2_ref,
                  westT_ref, qtotT_ref):
    f32 = jnp.float32
    sT = sT_ref[...]
    h1T = jnp.maximum(
        jnp.dot(hw1_w1T_ref[...], sT, preferred_element_type=f32) + hw1_b1_ref[...], 0.0)
    w1T = jnp.abs(
        jnp.dot(hw1_w2T_ref[...], h1T, preferred_element_type=f32) + hw1_b2_ref[...])
    hfT = jnp.maximum(
        jnp.dot(hwf_w1T_ref[...], sT, preferred_element_type=f32) + hwf_b1_ref[...], 0.0)
    wfT = jnp.abs(
        jnp.dot(hwf_w2T_ref[...], hfT, preferred_element_type=f32) + hwf_b2_ref[...])
    b1T = jnp.dot(hb1_wT_ref[...], sT, preferred_element_type=f32) + hb1_b_ref[...]
    hvT = jnp.maximum(
        jnp.dot(v_w1T_ref[...], sT, preferred_element_type=f32) + v_b1_ref[...], 0.0)
    vT = jnp.sum(hvT * v_w2_ref[...], axis=0, keepdims=True) + v_b2_ref[...]

    raT = raT_ref[...]
    Wq = wq_ref[...]
    R = raT.shape[1]
    acnvT = jnp.zeros((N_AGENTS, N_ACTIONS, R), f32)
    for q in range(N_AGENTS):
        wqi = Wq[q * N_AGENTS:(q + 1) * N_AGENTS, :]
        raq = raT[q * N_ACTIONS:(q + 1) * N_ACTIONS, :]
        acnvT = acnvT + wqi[:, None, :] * raq[None, :, :]
    acnvT = acnvT.reshape(N_AGENTS * N_ACTIONS, R)

    rows = []
    for i in range(N_AGENTS):
        acc = b1T
        for c in range(N_ACTIONS):
            m = jnp.broadcast_to(acnvT[i * N_ACTIONS + c:i * N_ACTIONS + c + 1, :], (EMBED, R))
            acc = acc + m * w1T[c * EMBED:(c + 1) * EMBED, :]
        for c in range(N_ACTIONS):
            cc = N_ACTIONS + c
            m = jnp.broadcast_to(raT[i * N_ACTIONS + c:i * N_ACTIONS + c + 1, :], (EMBED, R))
            acc = acc + m * w1T[cc * EMBED:(cc + 1) * EMBED, :]
        hid = jnp.where(acc > 0, acc, jnp.exp(jnp.minimum(acc, 0.0)) - 1.0)
        y_i = jnp.sum(hid * wfT, axis=0, keepdims=True) + vT
        rows.append(jnp.abs(y_i))
    westT = jnp.concatenate(rows, axis=0)
    westT_ref[...] = westT
    aqT = aqT_ref[...]
    qtot = jnp.sum(westT * aqT, axis=0, keepdims=True)
    qsum = jnp.sum(aqT, axis=0, keepdims=True)
    qsel = jnp.where(t_ref[...] != 0, qsum, qtot)
    qtotT_ref[...] = jnp.broadcast_to(qsel, (N_AGENTS, R))


def _coalition_weights(bs):
    with jax.ensure_compile_time_eval():
        perm = jnp.argsort(
            jax.random.uniform(jax.random.key(42), (bs * SAMPLE, N_AGENTS)), axis=-1)
        perm3 = perm.reshape(bs, SAMPLE, N_AGENTS)
        inv = jnp.argsort(perm3, axis=-1)
        mask = (inv[:, :, None, :] < perm3[:, :, :, None]).astype(jnp.float32)
        W = (perm3[:, :, :, None].astype(jnp.float32) * mask).sum(axis=1)
        W = W / (N_AGENTS * SAMPLE)
        W = W.transpose(2, 1, 0).reshape(N_AGENTS * N_AGENTS, bs)
    return W


def kernel(states, actions, agent_qs, max_filter, target,
           hw1_w1, hw1_b1, hw1_w2, hw1_b2,
           hwf_w1, hwf_b1, hwf_w2, hwf_b2,
           hb1_w, hb1_b, v_w1, v_b1, v_w2, v_b2):
    B0, T0 = states.shape[0], states.shape[1]
    bs = B0 * T0
    Wq = _coalition_weights(bs)

    sT = states.reshape(bs, STATE_DIM).T
    raT = actions.reshape(bs, N_AGENTS * N_ACTIONS).astype(jnp.float32).T
    aqT = agent_qs.reshape(bs, N_AGENTS).T
    tv = jnp.asarray(target, jnp.int32).reshape(1, 1)

    col = lambda i: (0, i)
    rep = lambda i: (0, 0)
    grid = (bs // BLK,)
    out = pl.pallas_call(
        _mixer_kernel,
        grid=grid,
        in_specs=[
            pl.BlockSpec((STATE_DIM, BLK), col),
            pl.BlockSpec((N_AGENTS * N_ACTIONS, BLK), col),
            pl.BlockSpec((N_AGENTS * N_AGENTS, BLK), col),
            pl.BlockSpec((N_AGENTS, BLK), col),
            pl.BlockSpec((1, 1), rep),
            pl.BlockSpec((256, STATE_DIM), rep),
            pl.BlockSpec((256, 1), rep),
            pl.BlockSpec((2 * N_ACTIONS * EMBED, 256), rep),
            pl.BlockSpec((2 * N_ACTIONS * EMBED, 1), rep),
            pl.BlockSpec((256, STATE_DIM), rep),
            pl.BlockSpec((256, 1), rep),
            pl.BlockSpec((EMBED, 256), rep),
            pl.BlockSpec((EMBED, 1), rep),
            pl.BlockSpec((EMBED, STATE_DIM), rep),
            pl.BlockSpec((EMBED, 1), rep),
            pl.BlockSpec((EMBED, STATE_DIM), rep),
            pl.BlockSpec((EMBED, 1), rep),
            pl.BlockSpec((EMBED, 1), rep),
            pl.BlockSpec((1, 1), rep),
        ],
        out_specs=[
            pl.BlockSpec((N_AGENTS, BLK), col),
            pl.BlockSpec((N_AGENTS, BLK), col),
        ],
        out_shape=[
            jax.ShapeDtypeStruct((N_AGENTS, bs), jnp.float32),
            jax.ShapeDtypeStruct((N_AGENTS, bs), jnp.float32),
        ],
    )(
        sT, raT, Wq, aqT, tv,
        hw1_w1.T, hw1_b1.reshape(-1, 1), hw1_w2.T, hw1_b2.reshape(-1, 1),
        hwf_w1.T, hwf_b1.reshape(-1, 1), hwf_w2.T, hwf_b2.reshape(-1, 1),
        hb1_w.T, hb1_b.reshape(-1, 1), v_w1.T, v_b1.reshape(-1, 1),
        v_w2, v_b2.reshape(1, 1),
    )
    w_est = out[0].T.reshape(B0, T0, N_AGENTS)
    q_tot = out[1][0].reshape(B0, T0, 1)
    return q_tot, w_est


# speedup vs baseline: 1.1148x; 1.1148x over previous
import jax
import jax.numpy as jnp
from jax.experimental import pallas as pl
from jax.experimental.pallas import tpu as pltpu

N_AGENTS = 8
N_ACTIONS = 16
STATE_DIM = 256
EMBED = 64
SAMPLE = 16
BLK = 256


def _mixer_kernel(sT_ref, raT_ref, wq_ref, aqT_ref,
                  hw1_w1T_ref, hw1_b1_ref, hw1_w2T_ref, hw1_b2_ref,
                  hwf_w1T_ref, hwf_b1_ref, hwf_w2T_ref, hwf_b2_ref,
                  hb1_wT_ref, hb1_b_ref, v_w1T_ref, v_b1_ref, v_w2_ref, v_b2_ref,
                  westT_ref, qtotT_ref):
    f32 = jnp.float32
    sT = sT_ref[...]
    h1T = jnp.maximum(
        jnp.dot(hw1_w1T_ref[...], sT, preferred_element_type=f32) + hw1_b1_ref[...], 0.0)
    w1T = jnp.abs(
        jnp.dot(hw1_w2T_ref[...], h1T, preferred_element_type=f32) + hw1_b2_ref[...])
    hfT = jnp.maximum(
        jnp.dot(hwf_w1T_ref[...], sT, preferred_element_type=f32) + hwf_b1_ref[...], 0.0)
    wfT = jnp.abs(
        jnp.dot(hwf_w2T_ref[...], hfT, preferred_element_type=f32) + hwf_b2_ref[...])
    b1T = jnp.dot(hb1_wT_ref[...], sT, preferred_element_type=f32) + hb1_b_ref[...]
    hvT = jnp.maximum(
        jnp.dot(v_w1T_ref[...], sT, preferred_element_type=f32) + v_b1_ref[...], 0.0)
    vT = jnp.sum(hvT * v_w2_ref[...], axis=0, keepdims=True) + v_b2_ref[...]

    raT = raT_ref[...]
    Wq = wq_ref[...]
    R = raT.shape[1]
    acnvT = jnp.zeros((N_AGENTS, N_ACTIONS, R), f32)
    for q in range(N_AGENTS):
        wqi = Wq[q * N_AGENTS:(q + 1) * N_AGENTS, :]
        raq = raT[q * N_ACTIONS:(q + 1) * N_ACTIONS, :]
        acnvT = acnvT + wqi[:, None, :] * raq[None, :, :]
    acnvT = acnvT.reshape(N_AGENTS * N_ACTIONS, R)

    rows = []
    for i in range(N_AGENTS):
        acc = b1T
        for c in range(N_ACTIONS):
            m = jnp.broadcast_to(acnvT[i * N_ACTIONS + c:i * N_ACTIONS + c + 1, :], (EMBED, R))
            acc = acc + m * w1T[c * EMBED:(c + 1) * EMBED, :]
        for c in range(N_ACTIONS):
            cc = N_ACTIONS + c
            m = jnp.broadcast_to(raT[i * N_ACTIONS + c:i * N_ACTIONS + c + 1, :], (EMBED, R))
            acc = acc + m * w1T[cc * EMBED:(cc + 1) * EMBED, :]
        hid = jnp.where(acc > 0, acc, jnp.exp(jnp.minimum(acc, 0.0)) - 1.0)
        y_i = jnp.sum(hid * wfT, axis=0, keepdims=True) + vT
        rows.append(jnp.abs(y_i))
    westT = jnp.concatenate(rows, axis=0)
    westT_ref[...] = westT
    qtot = jnp.sum(westT * aqT_ref[...], axis=0, keepdims=True)
    qtotT_ref[...] = jnp.broadcast_to(qtot, (N_AGENTS, R))


def _coalition_weights(bs):
    with jax.ensure_compile_time_eval():
        perm = jnp.argsort(
            jax.random.uniform(jax.random.key(42), (bs * SAMPLE, N_AGENTS)), axis=-1)
        perm3 = perm.reshape(bs, SAMPLE, N_AGENTS)
        inv = jnp.argsort(perm3, axis=-1)
        mask = (inv[:, :, None, :] < perm3[:, :, :, None]).astype(jnp.float32)
        W = (perm3[:, :, :, None].astype(jnp.float32) * mask).sum(axis=1)
        W = W / (N_AGENTS * SAMPLE)
        W = W.transpose(2, 1, 0).reshape(N_AGENTS * N_AGENTS, bs)
    return W


def kernel(states, actions, agent_qs, max_filter, target,
           hw1_w1, hw1_b1, hw1_w2, hw1_b2,
           hwf_w1, hwf_b1, hwf_w2, hwf_b2,
           hb1_w, hb1_b, v_w1, v_b1, v_w2, v_b2):
    B0, T0 = states.shape[0], states.shape[1]
    bs = B0 * T0
    Wq = _coalition_weights(bs)

    sT = states.reshape(bs, STATE_DIM).T
    raT = actions.reshape(bs, N_AGENTS * N_ACTIONS).astype(jnp.float32).T
    aqT = agent_qs.reshape(bs, N_AGENTS).T

    col = lambda i: (0, i)
    rep = lambda i: (0, 0)
    grid = (bs // BLK,)
    out = pl.pallas_call(
        _mixer_kernel,
        grid=grid,
        compiler_params=pltpu.CompilerParams(
            dimension_semantics=("parallel",)),
        in_specs=[
            pl.BlockSpec((STATE_DIM, BLK), col),
            pl.BlockSpec((N_AGENTS * N_ACTIONS, BLK), col),
            pl.BlockSpec((N_AGENTS * N_AGENTS, BLK), col),
            pl.BlockSpec((N_AGENTS, BLK), col),
            pl.BlockSpec((256, STATE_DIM), rep),
            pl.BlockSpec((256, 1), rep),
            pl.BlockSpec((2 * N_ACTIONS * EMBED, 256), rep),
            pl.BlockSpec((2 * N_ACTIONS * EMBED, 1), rep),
            pl.BlockSpec((256, STATE_DIM), rep),
            pl.BlockSpec((256, 1), rep),
            pl.BlockSpec((EMBED, 256), rep),
            pl.BlockSpec((EMBED, 1), rep),
            pl.BlockSpec((EMBED, STATE_DIM), rep),
            pl.BlockSpec((EMBED, 1), rep),
            pl.BlockSpec((EMBED, STATE_DIM), rep),
            pl.BlockSpec((EMBED, 1), rep),
            pl.BlockSpec((EMBED, 1), rep),
            pl.BlockSpec((1, 1), rep),
        ],
        out_specs=[
            pl.BlockSpec((N_AGENTS, BLK), col),
            pl.BlockSpec((N_AGENTS, BLK), col),
        ],
        out_shape=[
            jax.ShapeDtypeStruct((N_AGENTS, bs), jnp.float32),
            jax.ShapeDtypeStruct((N_AGENTS, bs), jnp.float32),
        ],
    )(
        sT, raT, Wq, aqT,
        hw1_w1.T, hw1_b1.reshape(-1, 1), hw1_w2.T, hw1_b2.reshape(-1, 1),
        hwf_w1.T, hwf_b1.reshape(-1, 1), hwf_w2.T, hwf_b2.reshape(-1, 1),
        hb1_w.T, hb1_b.reshape(-1, 1), v_w1.T, v_b1.reshape(-1, 1),
        v_w2, v_b2.reshape(1, 1),
    )
    w_est = out[0].T.reshape(B0, T0, N_AGENTS)
    q_tot = out[1][0].reshape(B0, T0, 1)
    q_tot = jnp.where(target != 0,
                      jnp.sum(agent_qs, axis=2, keepdims=True), q_tot)
    return q_tot, w_est
